# baseline (speedup 1.0000x reference)
import functools

import jax
import jax.numpy as jnp
from jax import lax
from jax.experimental import pallas as pl
from jax.experimental.pallas import tpu as pltpu
from jax.experimental.pallas import tpu_sc as plsc

_TILE_V = 4096


def _sc_gather_t(tT, idx):
    D, V = tT.shape
    B = idx.shape[0]
    info = plsc.get_sparse_core_info()
    nc, ns = info.num_cores, info.num_subcores
    nw = nc * ns
    bw = B // nw
    bc = B // nc
    chunk = 8
    mesh = plsc.VectorSubcoreMesh(core_axis_name="c", subcore_axis_name="s")

    @functools.partial(
        pl.kernel,
        mesh=mesh,
        out_type=jax.ShapeDtypeStruct((B, D), jnp.float32),
        scratch_types=[
            pltpu.VMEM((bw,), jnp.int32),
            pltpu.VMEM((chunk, D, 128), jnp.float32),
            pltpu.VMEM((bw, D), jnp.float32),
            pltpu.VMEM_SHARED((bc, D), jnp.float32),
            pltpu.SemaphoreType.DMA,
        ],
        compiler_params=pltpu.CompilerParams(needs_layout_passes=False),
    )
    def gather_k(tT_hbm, idx_hbm, out_hbm, idx_v, g_v, h_v, stage_v, sem):
        cid = lax.axis_index("c")
        sid = lax.axis_index("s")
        wid = cid * ns + sid
        base = wid * bw
        pltpu.sync_copy(idx_hbm.at[pl.ds(base, bw)], idx_v)
        plsc.subcore_barrier()
        iota = lax.broadcasted_iota(jnp.int32, (16,), 0)
        for c0 in range(0, bw, chunk):
            copies = []
            lanes = []
            for j in range(chunk):
                i = c0 + j
                v = idx_v[pl.ds((i // 16) * 16, 16)]
                m = (iota == (i % 16)).astype(jnp.int32)
                r = jnp.sum(v * m)
                start = pl.multiple_of(jnp.bitwise_and(r, jnp.int32(-128)), 128)
                lanes.append(jnp.bitwise_and(r, jnp.int32(127)))
                copies.append(pltpu.async_copy(
                    tT_hbm.at[:, pl.ds(start, 128)], g_v.at[j], sem))
            for cp in copies:
                cp.wait()
            plsc.subcore_barrier()
            for j in range(chunk):
                i = c0 + j
                d0 = jnp.broadcast_to(jnp.int32(j), (16,))
                lane_vec = jnp.broadcast_to(lanes[j], (16,))
                for g in range(D // 16):
                    d1 = iota + g * 16
                    hv = plsc.load_gather(g_v, [d0, d1, lane_vec])
                    h_v[i, pl.ds(g * 16, 16)] = hv
        pltpu.sync_copy(h_v, stage_v.at[pl.ds(sid * bw, bw), :])
        plsc.subcore_barrier()

        @pl.when(sid == 0)
        def _():
            pltpu.sync_copy(stage_v, out_hbm.at[pl.ds(cid * bc, bc), :])

    return gather_k(tT, idx)


_TC_WPS = 8


def _tc_gather_t(tT, tcol, lane):
    D, V = tT.shape
    Bh = tcol.shape[0]
    grid = Bh // _TC_WPS

    def body(tcol_ref, lane_ref, *refs):
        o_ref = refs[-1]
        i = pl.program_id(0)
        for k in range(_TC_WPS):
            l = lane_ref[i * _TC_WPS + k]
            onehot = (lax.broadcasted_iota(jnp.int32, (1, 128), 1) == l
                      ).astype(jnp.float32)
            row = lax.dot_general(
                onehot, refs[k][...],
                (((1,), (1,)), ((), ())),
                preferred_element_type=jnp.float32,
            )
            o_ref[pl.ds(k, 1), :] = row

    def win_spec(k):
        return pl.BlockSpec(
            (D, 128),
            lambda i, tcol_ref, lane_ref, k=k: (0, tcol_ref[i * _TC_WPS + k]))

    return pl.pallas_call(
        body,
        grid_spec=pltpu.PrefetchScalarGridSpec(
            num_scalar_prefetch=2,
            grid=(grid,),
            in_specs=[win_spec(k) for k in range(_TC_WPS)],
            out_specs=pl.BlockSpec((_TC_WPS, D), lambda i, *_: (i, 0)),
        ),
        out_shape=jax.ShapeDtypeStruct((Bh, D), jnp.float32),
    )(tcol, lane, *([tT] * _TC_WPS))


def _decoder_body(h0_ref, h1_ref, wt_ref, o_ref):
    half = h0_ref.shape[0]
    wt = wt_ref[...]
    for k, h_ref in enumerate((h0_ref, h1_ref)):
        logits = lax.dot_general(
            wt, h_ref[...],
            (((0,), (1,)), ((), ())),
            preferred_element_type=jnp.float32,
        )
        o_ref[:, pl.ds(k * half, half)] = 0.5 * jnp.tanh(0.5 * logits) + 0.5


def kernel(x, emb_table, decoder_W):
    B = x.shape[0]
    V, D = decoder_W.shape
    half = B // 2
    x = x.astype(jnp.int32)
    tT = emb_table.T
    h0 = _sc_gather_t(tT, x[:half])
    x1 = x[half:]
    h1 = _tc_gather_t(tT, x1 >> 7, x1 & 127)
    grid = pl.cdiv(V, _TILE_V)
    outT = pl.pallas_call(
        _decoder_body,
        grid=(grid,),
        in_specs=[
            pl.BlockSpec((half, D), lambda j: (0, 0)),
            pl.BlockSpec((half, D), lambda j: (0, 0)),
            pl.BlockSpec((D, _TILE_V), lambda j: (0, j)),
        ],
        out_specs=pl.BlockSpec((_TILE_V, B), lambda j: (j, 0)),
        out_shape=jax.ShapeDtypeStruct((V, B), jnp.float32),
    )(h0, h1, decoder_W.T)
    return outT.T


# speedup vs baseline: 1.1204x; 1.1204x over previous
import functools

import jax
import jax.numpy as jnp
from jax import lax
from jax.experimental import pallas as pl
from jax.experimental.pallas import tpu as pltpu
from jax.experimental.pallas import tpu_sc as plsc

_TILE_V = 4096


def _sc_gather_t(tT, idx):
    D, V = tT.shape
    B = idx.shape[0]
    info = plsc.get_sparse_core_info()
    nc, ns = info.num_cores, info.num_subcores
    nw = nc * ns
    bw = B // nw
    bc = B // nc
    chunk = 8
    mesh = plsc.VectorSubcoreMesh(core_axis_name="c", subcore_axis_name="s")

    @functools.partial(
        pl.kernel,
        mesh=mesh,
        out_type=jax.ShapeDtypeStruct((B, D), jnp.float32),
        scratch_types=[
            pltpu.VMEM((bw,), jnp.int32),
            pltpu.VMEM((chunk, D, 128), jnp.float32),
            pltpu.VMEM((bw, D), jnp.float32),
            pltpu.VMEM_SHARED((bc, D), jnp.float32),
            pltpu.SemaphoreType.DMA,
        ],
        compiler_params=pltpu.CompilerParams(needs_layout_passes=False),
    )
    def gather_k(tT_hbm, idx_hbm, out_hbm, idx_v, g_v, h_v, stage_v, sem):
        cid = lax.axis_index("c")
        sid = lax.axis_index("s")
        wid = cid * ns + sid
        base = wid * bw
        pltpu.sync_copy(idx_hbm.at[pl.ds(base, bw)], idx_v)
        plsc.subcore_barrier()
        iota = lax.broadcasted_iota(jnp.int32, (16,), 0)
        for c0 in range(0, bw, chunk):
            copies = []
            lanes = []
            for j in range(chunk):
                i = c0 + j
                v = idx_v[pl.ds((i // 16) * 16, 16)]
                m = (iota == (i % 16)).astype(jnp.int32)
                r = jnp.sum(v * m)
                start = pl.multiple_of(jnp.bitwise_and(r, jnp.int32(-128)), 128)
                lanes.append(jnp.bitwise_and(r, jnp.int32(127)))
                copies.append(pltpu.async_copy(
                    tT_hbm.at[:, pl.ds(start, 128)], g_v.at[j], sem))
            for cp in copies:
                cp.wait()
            plsc.subcore_barrier()
            for j in range(chunk):
                i = c0 + j
                d0 = jnp.broadcast_to(jnp.int32(j), (16,))
                lane_vec = jnp.broadcast_to(lanes[j], (16,))
                for g in range(D // 16):
                    d1 = iota + g * 16
                    hv = plsc.load_gather(g_v, [d0, d1, lane_vec])
                    h_v[i, pl.ds(g * 16, 16)] = hv
        pltpu.sync_copy(h_v, stage_v.at[pl.ds(sid * bw, bw), :])
        plsc.subcore_barrier()

        @pl.when(sid == 0)
        def _():
            pltpu.sync_copy(stage_v, out_hbm.at[pl.ds(cid * bc, bc), :])

    return gather_k(tT, idx)


def _decoder_body(h_ref, wt_ref, o_ref):
    logits = lax.dot_general(
        wt_ref[...], h_ref[...],
        (((0,), (1,)), ((), ())),
        preferred_element_type=jnp.float32,
    )
    o_ref[...] = 0.5 * jnp.tanh(0.5 * logits) + 0.5


def kernel(x, emb_table, decoder_W):
    B = x.shape[0]
    V, D = decoder_W.shape
    h = _sc_gather_t(emb_table.T, x.astype(jnp.int32))
    grid = pl.cdiv(V, _TILE_V)
    outT = pl.pallas_call(
        _decoder_body,
        grid=(grid,),
        in_specs=[
            pl.BlockSpec((B, D), lambda j: (0, 0)),
            pl.BlockSpec((D, _TILE_V), lambda j: (0, j)),
        ],
        out_specs=pl.BlockSpec((_TILE_V, B), lambda j: (j, 0)),
        out_shape=jax.ShapeDtypeStruct((V, B), jnp.float32),
    )(h, decoder_W.T)
    return outT.T
